# Initial kernel scaffold; baseline (speedup 1.0000x reference)
import jax
import jax.numpy as jnp
from jax.experimental import pallas as pl


def kernel(pos_ids, table):
    raise NotImplementedError("write your pallas kernel here")


# speedup vs baseline: 2.2622x; 2.2622x over previous
import functools

import jax
import jax.numpy as jnp
from jax import lax
from jax.experimental import pallas as pl
from jax.experimental.pallas import tpu as pltpu
from jax.experimental.pallas import tpu_sc as plsc

_DIM = 1024
_NC = 2
_NS = 16
_NW = _NC * _NS
_CHUNK = 32


def _emb_body(total, bpw, nchunk,
              idx_hbm, table_hbm, out_hbm,
              idx_v, rows0, rows1, gs0, gs1, ss0, ss1):
    wid = lax.axis_index("s") * _NC + lax.axis_index("c")
    base = wid * bpw

    pltpu.sync_copy(idx_hbm.at[pl.ds(base, bpw)], idx_v)

    def gather(chunk, buf, sem):
        src = table_hbm.at[idx_v.at[pl.ds(chunk * _CHUNK, _CHUNK)]]
        return pltpu.make_async_copy(src, buf, sem)

    def store(chunk, buf, sem):
        dst = out_hbm.at[pl.ds(base + chunk * _CHUNK, _CHUNK)]
        return pltpu.make_async_copy(buf, dst, sem)

    gather(0, rows0, gs0).start()
    gather(1, rows1, gs1).start()

    npair = nchunk // 2

    def pair(p, _):
        c0 = 2 * p
        c1 = c0 + 1
        gather(c0, rows0, gs0).wait()
        store(c0, rows0, ss0).start()
        gather(c1, rows1, gs1).wait()
        store(c1, rows1, ss1).start()

        @pl.when(p + 1 < npair)
        def _():
            store(c0, rows0, ss0).wait()
            gather(c0 + 2, rows0, gs0).start()
            store(c1, rows1, ss1).wait()
            gather(c1 + 2, rows1, gs1).start()

        return None

    lax.fori_loop(0, npair, pair, None, unroll=False)

    store(nchunk - 2, rows0, ss0).wait()
    store(nchunk - 1, rows1, ss1).wait()


def kernel(pos_ids, table):
    batch, seq = pos_ids.shape
    dim = table.shape[1]
    total = batch * seq
    bpw = total // _NW
    nchunk = bpw // _CHUNK

    flat_ids = pos_ids.reshape(total).astype(jnp.int32)

    mesh = plsc.VectorSubcoreMesh(core_axis_name="c", subcore_axis_name="s")
    body = functools.partial(_emb_body, total, bpw, nchunk)
    out = pl.kernel(
        body,
        out_type=jax.ShapeDtypeStruct((total, dim), jnp.float32),
        mesh=mesh,
        scratch_types=[
            pltpu.VMEM((bpw,), jnp.int32),
            pltpu.VMEM((_CHUNK, dim), jnp.float32),
            pltpu.VMEM((_CHUNK, dim), jnp.float32),
            pltpu.SemaphoreType.DMA,
            pltpu.SemaphoreType.DMA,
            pltpu.SemaphoreType.DMA,
            pltpu.SemaphoreType.DMA,
        ],
    )(flat_ids, table)
    return out.reshape(batch, seq, dim)


# speedup vs baseline: 2.3646x; 1.0453x over previous
import functools

import jax
import jax.numpy as jnp
from jax import lax
from jax.experimental import pallas as pl
from jax.experimental.pallas import tpu as pltpu
from jax.experimental.pallas import tpu_sc as plsc

_DIM = 1024
_NC = 2
_NS = 16
_NW = _NC * _NS
_CHUNK = 32


def _emb_body(total, bpw, nchunk,
              idx_hbm, table_hbm, out_hbm,
              idx_v, rows, gs, ss):
    wid = lax.axis_index("s") * _NC + lax.axis_index("c")
    base = wid * bpw

    pltpu.sync_copy(idx_hbm.at[pl.ds(base, bpw)], idx_v)

    def gather(chunk, b):
        src = table_hbm.at[idx_v.at[pl.ds(chunk * _CHUNK, _CHUNK)]]
        return pltpu.make_async_copy(src, rows[b], gs[b])

    def store(chunk, b):
        dst = out_hbm.at[pl.ds(base + chunk * _CHUNK, _CHUNK)]
        return pltpu.make_async_copy(rows[b], dst, ss[b])

    gather(0, 0).start()
    gather(1, 1).start()

    ngroup = (nchunk - 2) // 3

    def group(g, _):
        for j in range(3):
            c = 3 * g + j
            bn = (j + 2) % 3
            gather(c, j).wait()
            store(c, j).start()

            @pl.when(c >= 1)
            def _():
                store(c - 1, bn).wait()

            gather(c + 2, bn).start()
        return None

    lax.fori_loop(0, ngroup, group, None, unroll=False)

    for c in range(3 * ngroup, nchunk):
        b = c % 3
        gather(c, b).wait()
        store(c - 1, (b + 2) % 3).wait()
        store(c, b).start()
    store(nchunk - 1, (nchunk - 1) % 3).wait()


def kernel(pos_ids, table):
    batch, seq = pos_ids.shape
    dim = table.shape[1]
    total = batch * seq
    bpw = total // _NW
    nchunk = bpw // _CHUNK

    flat_ids = pos_ids.reshape(total).astype(jnp.int32)

    mesh = plsc.VectorSubcoreMesh(core_axis_name="c", subcore_axis_name="s")
    body = functools.partial(_emb_body, total, bpw, nchunk)
    out = pl.kernel(
        body,
        out_type=jax.ShapeDtypeStruct((total, dim), jnp.float32),
        mesh=mesh,
        scratch_types=[
            pltpu.VMEM((bpw,), jnp.int32),
            [pltpu.VMEM((_CHUNK, dim), jnp.float32) for _ in range(3)],
            [pltpu.SemaphoreType.DMA for _ in range(3)],
            [pltpu.SemaphoreType.DMA for _ in range(3)],
        ],
    )(flat_ids, table)
    return out.reshape(batch, seq, dim)
